# baseline (speedup 1.0000x reference)
import jax
import jax.numpy as jnp
from jax.experimental import pallas as pl
from jax.experimental.pallas import tpu as pltpu

_BT = 1024


def _router_block(x_ref, w_ref, b_ref, o_ref):
    logits = jax.lax.dot_general(
        x_ref[...].astype(jnp.bfloat16), w_ref[...].astype(jnp.bfloat16),
        dimension_numbers=(((1,), (1,)), ((), ())),
        preferred_element_type=jnp.float32) + b_ref[...]
    m = jnp.max(logits, axis=-1, keepdims=True)
    e = jnp.exp(logits - m)
    o_ref[...] = e * (1.0 / jnp.sum(e, axis=-1, keepdims=True))


def kernel(x, W, b):
    nt, h = x.shape
    ne = W.shape[0]
    b2 = b.reshape(1, ne)
    grid = (nt // _BT,)
    return pl.pallas_call(
        _router_block,
        grid=grid,
        in_specs=[
            pl.BlockSpec((_BT, h), lambda i: (i, 0)),
            pl.BlockSpec((ne, h), lambda i: (0, 0)),
            pl.BlockSpec((1, ne), lambda i: (0, 0)),
        ],
        out_specs=pl.BlockSpec((_BT, ne), lambda i: (i, 0)),
        out_shape=jax.ShapeDtypeStruct((nt, ne), jnp.float32),
        compiler_params=pltpu.CompilerParams(
            dimension_semantics=("parallel",)),
    )(x, W, b2)


# speedup vs baseline: 1.0008x; 1.0008x over previous
import jax
import jax.numpy as jnp
from jax.experimental import pallas as pl

_BT = 1024


def _router_block(x_ref, w_ref, b_ref, o_ref):
    logits = jax.lax.dot_general(
        x_ref[...], w_ref[...],
        dimension_numbers=(((1,), (1,)), ((), ())),
        preferred_element_type=jnp.float32) + b_ref[...]
    m = jnp.max(logits, axis=-1, keepdims=True)
    e = jnp.exp(logits - m)
    o_ref[...] = e * (1.0 / jnp.sum(e, axis=-1, keepdims=True))


def kernel(x, W, b):
    nt, h = x.shape
    ne = W.shape[0]
    b2 = b.reshape(1, ne)
    grid = (nt // _BT,)
    return pl.pallas_call(
        _router_block,
        grid=grid,
        in_specs=[
            pl.BlockSpec((_BT, h), lambda i: (i, 0)),
            pl.BlockSpec((ne, h), lambda i: (0, 0)),
            pl.BlockSpec((1, ne), lambda i: (0, 0)),
        ],
        out_specs=pl.BlockSpec((_BT, ne), lambda i: (i, 0)),
        out_shape=jax.ShapeDtypeStruct((nt, ne), jnp.float32),
    )(x, W, b2)
